# Initial kernel scaffold; baseline (speedup 1.0000x reference)
import jax
import jax.numpy as jnp
from jax.experimental import pallas as pl


def kernel(x_user, x_item, Wn0_u2i, b0_u2i, Wr0_u2i, Wn0_i2u, b0_i2u, Wr0_i2u, Wn1_u2i, b1_u2i, Wr1_u2i, Wn1_i2u, b1_i2u, Wr1_i2u, ei_u2i, ei_i2u):
    raise NotImplementedError("write your pallas kernel here")


# speedup vs baseline: 7.1401x; 7.1401x over previous
import functools

import jax
import jax.numpy as jnp
from jax import lax
from jax.experimental import pallas as pl
from jax.experimental.pallas import tpu as pltpu
from jax.experimental.pallas import tpu_sc as plsc

N = 10000
NP = 10240
E = 160000
D = 128
C = 16
CH = 125
ROWS = E // CH
NSC = 2
NTILE = 16
RPT = ROWS // NTILE
RPT2 = ROWS // (2 * NTILE)
STRIPE = NP // NTILE
ZR = 32

_mesh = plsc.VectorSubcoreMesh(
    core_axis_name="c", subcore_axis_name="s",
    num_cores=NSC, num_subcores=NTILE)

_DN = (((1,), (1,)), ((), ()))

_BLK = 1024


def _tc0_body(xu, xi, wn_u2i, wr_u2i, wn_i2u, wr_i2u, b_u2i, b_i2u,
              m_u2i, r_i, m_i2u, r_u):
    xu_v = xu[...]
    xi_v = xi[...]
    m_u2i[...] = lax.dot_general(xu_v, wn_u2i[...], _DN,
                                 preferred_element_type=jnp.float32)
    r_i[...] = lax.dot_general(xi_v, wr_u2i[...], _DN,
                               preferred_element_type=jnp.float32) + b_u2i[...]
    m_i2u[...] = lax.dot_general(xi_v, wn_i2u[...], _DN,
                                 preferred_element_type=jnp.float32)
    r_u[...] = lax.dot_general(xu_v, wr_i2u[...], _DN,
                               preferred_element_type=jnp.float32) + b_i2u[...]


def _tc0(xu, xi, wn_u2i, wr_u2i, wn_i2u, wr_i2u, b_u2i, b_i2u):
    row = pl.BlockSpec((_BLK, D), lambda i: (i, 0))
    w = pl.BlockSpec((D, D), lambda i: (0, 0))
    b = pl.BlockSpec((1, D), lambda i: (0, 0))
    return pl.pallas_call(
        _tc0_body,
        grid=(NP // _BLK,),
        in_specs=[row, row, w, w, w, w, b, b],
        out_specs=[row, row, row, row],
        out_shape=[jax.ShapeDtypeStruct((NP, D), jnp.float32)] * 4,
    )(xu, xi, wn_u2i, wr_u2i, wn_i2u, wr_i2u, b_u2i, b_i2u)


def _tc1_body(acc_i, cnt_i, root_i, acc_u, cnt_u, root_u, wr1, b1,
              h_i, root1):
    ci = jnp.maximum(cnt_i[...], 1.0)
    h_i[...] = jnp.maximum(acc_i[...] / ci + root_i[...], 0.0)
    cu = jnp.maximum(cnt_u[...], 1.0)
    h_u = jnp.maximum(acc_u[...] / cu + root_u[...], 0.0)
    root1[...] = lax.dot_general(h_u, wr1[...], _DN,
                                 preferred_element_type=jnp.float32) + b1[...]


def _tc1(acc_i, cnt_i, root_i, acc_u, cnt_u, root_u, wr1, b1):
    row = pl.BlockSpec((_BLK, D), lambda i: (i, 0))
    cnt = pl.BlockSpec((_BLK, 1), lambda i: (i, 0))
    w = pl.BlockSpec((C, D), lambda i: (0, 0))
    b = pl.BlockSpec((1, C), lambda i: (0, 0))
    out = pl.BlockSpec((_BLK, C), lambda i: (i, 0))
    return pl.pallas_call(
        _tc1_body,
        grid=(NP // _BLK,),
        in_specs=[row, cnt, row, row, cnt, row, w, b],
        out_specs=[row, out],
        out_shape=[jax.ShapeDtypeStruct((NP, D), jnp.float32),
                   jax.ShapeDtypeStruct((NP, C), jnp.float32)],
    )(acc_i, cnt_i, root_i, acc_u, cnt_u, root_u, wr1, b1)


def _tc2_body(acc_a, acc_b, cnt_u, root1, wn1, out):
    cu = jnp.maximum(cnt_u[...], 1.0)
    mean = (acc_a[...] + acc_b[...]) / cu
    out[...] = lax.dot_general(mean, wn1[...], _DN,
                               preferred_element_type=jnp.float32) + root1[...]


def _tc2(acc_a, acc_b, cnt_u, root1, wn1):
    row = pl.BlockSpec((_BLK, D), lambda i: (i, 0))
    blk = pl.BlockSpec((_BLK, C), lambda i: (i, 0))
    cnt = pl.BlockSpec((_BLK, 1), lambda i: (i, 0))
    w = pl.BlockSpec((C, D), lambda i: (0, 0))
    return pl.pallas_call(
        _tc2_body,
        grid=(NP // _BLK,),
        in_specs=[row, row, cnt, blk, w],
        out_specs=blk,
        out_shape=jax.ShapeDtypeStruct((NP, C), jnp.float32),
    )(acc_a, acc_b, cnt_u, root1, wn1)


def _fill2d(ref, nrows, nvecs, val):
    def body(i, _):
        for j in range(nvecs):
            ref[i, pl.ds(j * 16, 16)] = jnp.full((16,), val, jnp.float32)
        return 0
    lax.fori_loop(0, nrows, body, 0)


def _fill1d(ref, nvecs, val):
    def body(i, _):
        ref[pl.ds(i * 16, 16)] = jnp.full((16,), val, jnp.float32)
        return 0
    lax.fori_loop(0, nvecs, body, 0)


@functools.partial(
    pl.kernel,
    out_type=(
        jax.ShapeDtypeStruct((NP, D), jnp.float32),
        jax.ShapeDtypeStruct((NP,), jnp.float32),
        jax.ShapeDtypeStruct((NP, D), jnp.float32),
        jax.ShapeDtypeStruct((NP,), jnp.float32),
    ),
    mesh=_mesh,
    scratch_types=[
        pltpu.VMEM((RPT, CH), jnp.int32),
        pltpu.VMEM((RPT, CH), jnp.int32),
        pltpu.VMEM((CH, D), jnp.float32),
        pltpu.VMEM((128,), jnp.float32),
        pltpu.VMEM((ZR, D), jnp.float32),
        pltpu.VMEM((STRIPE,), jnp.float32),
        pltpu.VMEM_SHARED((NP, D), jnp.float32),
        pltpu.VMEM_SHARED((NP,), jnp.float32),
    ],
)
def _sc_layer0(m_u2i, s_u2i, d_u2i, m_i2u, s_i2u, d_i2u,
               acc_i, cnt_i, acc_u, cnt_u,
               idx_s, idx_d, rows, ones, zrow, zcnt, acc_sh, cnt_sh):
    cid = lax.axis_index("c")
    sid = lax.axis_index("s")
    base = sid * STRIPE

    _fill1d(ones, 128 // 16, 1.0)
    _fill2d(zrow, ZR, D // 16, 0.0)
    _fill1d(zcnt, STRIPE // 16, 0.0)
    for k in range(STRIPE // ZR):
        pltpu.sync_copy(zrow, acc_sh.at[pl.ds(base + k * ZR, ZR)])
    pltpu.sync_copy(zcnt, cnt_sh.at[pl.ds(base, STRIPE)])
    plsc.subcore_barrier()

    def do_rel(msg, s2d, d2d):
        rbase = sid * RPT
        pltpu.sync_copy(s2d.at[pl.ds(rbase, RPT)], idx_s)
        pltpu.sync_copy(d2d.at[pl.ds(rbase, RPT)], idx_d)

        def chunk(j, _):
            pltpu.sync_copy(msg.at[idx_s.at[j]], rows)
            pltpu.sync_copy(rows, acc_sh.at[idx_d.at[j]], add=True)
            pltpu.sync_copy(ones.at[pl.ds(0, CH)], cnt_sh.at[idx_d.at[j]],
                            add=True)
            return 0
        lax.fori_loop(0, RPT, chunk, 0)

    @pl.when(cid == 0)
    def _():
        do_rel(m_u2i, s_u2i, d_u2i)

    @pl.when(cid == 1)
    def _():
        do_rel(m_i2u, s_i2u, d_i2u)

    plsc.subcore_barrier()

    @pl.when(cid == 0)
    def _():
        pltpu.sync_copy(acc_sh.at[pl.ds(base, STRIPE)],
                        acc_i.at[pl.ds(base, STRIPE)])
        pltpu.sync_copy(cnt_sh.at[pl.ds(base, STRIPE)],
                        cnt_i.at[pl.ds(base, STRIPE)])

    @pl.when(cid == 1)
    def _():
        pltpu.sync_copy(acc_sh.at[pl.ds(base, STRIPE)],
                        acc_u.at[pl.ds(base, STRIPE)])
        pltpu.sync_copy(cnt_sh.at[pl.ds(base, STRIPE)],
                        cnt_u.at[pl.ds(base, STRIPE)])


@functools.partial(
    pl.kernel,
    out_type=(
        jax.ShapeDtypeStruct((NP, D), jnp.float32),
        jax.ShapeDtypeStruct((NP, D), jnp.float32),
    ),
    mesh=_mesh,
    scratch_types=[
        pltpu.VMEM((RPT2, CH), jnp.int32),
        pltpu.VMEM((RPT2, CH), jnp.int32),
        pltpu.VMEM((CH, D), jnp.float32),
        pltpu.VMEM((ZR, D), jnp.float32),
        pltpu.VMEM_SHARED((NP, D), jnp.float32),
    ],
)
def _sc_layer1(msg1, s_i2u, d_i2u, acc_a, acc_b,
               idx_s, idx_d, rows, z16, acc_sh):
    cid = lax.axis_index("c")
    sid = lax.axis_index("s")
    base = sid * STRIPE

    _fill2d(z16, ZR, D // 16, 0.0)
    for k in range(STRIPE // ZR):
        pltpu.sync_copy(z16, acc_sh.at[pl.ds(base + k * ZR, ZR)])
    plsc.subcore_barrier()

    rbase = (cid * NTILE + sid) * RPT2
    pltpu.sync_copy(s_i2u.at[pl.ds(rbase, RPT2)], idx_s)
    pltpu.sync_copy(d_i2u.at[pl.ds(rbase, RPT2)], idx_d)

    def chunk(j, _):
        pltpu.sync_copy(msg1.at[idx_s.at[j]], rows)
        pltpu.sync_copy(rows, acc_sh.at[idx_d.at[j]], add=True)
        return 0
    lax.fori_loop(0, RPT2, chunk, 0)
    plsc.subcore_barrier()

    @pl.when(cid == 0)
    def _():
        pltpu.sync_copy(acc_sh.at[pl.ds(base, STRIPE)],
                        acc_a.at[pl.ds(base, STRIPE)])

    @pl.when(cid == 1)
    def _():
        pltpu.sync_copy(acc_sh.at[pl.ds(base, STRIPE)],
                        acc_b.at[pl.ds(base, STRIPE)])


def _pad_rows(x):
    return jnp.concatenate(
        [x, jnp.zeros((NP - N,) + x.shape[1:], x.dtype)], axis=0)


def kernel(x_user, x_item, Wn0_u2i, b0_u2i, Wr0_u2i, Wn0_i2u, b0_i2u, Wr0_i2u,
           Wn1_u2i, b1_u2i, Wr1_u2i, Wn1_i2u, b1_i2u, Wr1_i2u,
           ei_u2i, ei_i2u):
    s_u2i = ei_u2i[0].reshape(ROWS, CH)
    d_u2i = ei_u2i[1].reshape(ROWS, CH)
    s_i2u = ei_i2u[0].reshape(ROWS, CH)
    d_i2u = ei_i2u[1].reshape(ROWS, CH)

    m_u2i, r_i, m_i2u, r_u = _tc0(
        _pad_rows(x_user), _pad_rows(x_item),
        Wn0_u2i, Wr0_u2i, Wn0_i2u, Wr0_i2u,
        b0_u2i.reshape(1, D), b0_i2u.reshape(1, D))

    acc_i, cnt_i, acc_u, cnt_u = _sc_layer0(
        m_u2i, s_u2i, d_u2i, m_i2u, s_i2u, d_i2u)

    h_i, root1 = _tc1(acc_i, cnt_i.reshape(NP, 1), r_i,
                      acc_u, cnt_u.reshape(NP, 1), r_u,
                      Wr1_i2u, b1_i2u.reshape(1, C))

    acc_a, acc_b = _sc_layer1(h_i, s_i2u, d_i2u)

    return _tc2(acc_a, acc_b, cnt_u.reshape(NP, 1), root1, Wn1_i2u)[:N]


# speedup vs baseline: 10.1990x; 1.4284x over previous
import functools

import jax
import jax.numpy as jnp
from jax import lax
from jax.experimental import pallas as pl
from jax.experimental.pallas import tpu as pltpu
from jax.experimental.pallas import tpu_sc as plsc

N = 10000
NP = 10240
E = 160000
D = 128
C = 16
CH = 128
ROWS = 1280
EP = ROWS * CH
NSC = 2
NTILE = 16
RPT = ROWS // NTILE
HALF = RPT // 2
RPT2 = ROWS // (2 * NTILE)
STRIPE = NP // NTILE

_mesh = plsc.VectorSubcoreMesh(
    core_axis_name="c", subcore_axis_name="s",
    num_cores=NSC, num_subcores=NTILE)

_DN = (((1,), (1,)), ((), ()))

_BLK = 1024


def _tc0_body(xu, xi, wn_u2i, wr_u2i, wn_i2u, wr_i2u, b_u2i, b_i2u,
              m_u2i, r_i, m_i2u, r_u):
    xu_v = xu[...]
    xi_v = xi[...]
    m_u2i[...] = lax.dot_general(xu_v, wn_u2i[...], _DN,
                                 preferred_element_type=jnp.float32)
    r_i[...] = lax.dot_general(xi_v, wr_u2i[...], _DN,
                               preferred_element_type=jnp.float32) + b_u2i[...]
    m_i2u[...] = lax.dot_general(xi_v, wn_i2u[...], _DN,
                                 preferred_element_type=jnp.float32)
    r_u[...] = lax.dot_general(xu_v, wr_i2u[...], _DN,
                               preferred_element_type=jnp.float32) + b_i2u[...]


def _tc0(xu, xi, wn_u2i, wr_u2i, wn_i2u, wr_i2u, b_u2i, b_i2u):
    row = pl.BlockSpec((_BLK, D), lambda i: (i, 0))
    w = pl.BlockSpec((D, D), lambda i: (0, 0))
    b = pl.BlockSpec((1, D), lambda i: (0, 0))
    return pl.pallas_call(
        _tc0_body,
        grid=(NP // _BLK,),
        in_specs=[row, row, w, w, w, w, b, b],
        out_specs=[row, row, row, row],
        out_shape=[jax.ShapeDtypeStruct((NP, D), jnp.float32)] * 4,
    )(xu, xi, wn_u2i, wr_u2i, wn_i2u, wr_i2u, b_u2i, b_i2u)


def _tc1_body(acc_i, cnt_i, root_i, acc_u, cnt_u, root_u, wr1, b1,
              h_i, root1):
    ci = jnp.maximum(cnt_i[...], 1.0)
    h_i[...] = jnp.maximum(acc_i[...] / ci + root_i[...], 0.0)
    cu = jnp.maximum(cnt_u[...], 1.0)
    h_u = jnp.maximum(acc_u[...] / cu + root_u[...], 0.0)
    root1[...] = lax.dot_general(h_u, wr1[...], _DN,
                                 preferred_element_type=jnp.float32) + b1[...]


def _tc1(acc_i, cnt_i, root_i, acc_u, cnt_u, root_u, wr1, b1):
    row = pl.BlockSpec((_BLK, D), lambda i: (i, 0))
    cnt = pl.BlockSpec((_BLK, 1), lambda i: (i, 0))
    w = pl.BlockSpec((C, D), lambda i: (0, 0))
    b = pl.BlockSpec((1, C), lambda i: (0, 0))
    out = pl.BlockSpec((_BLK, C), lambda i: (i, 0))
    return pl.pallas_call(
        _tc1_body,
        grid=(NP // _BLK,),
        in_specs=[row, cnt, row, row, cnt, row, w, b],
        out_specs=[row, out],
        out_shape=[jax.ShapeDtypeStruct((NP, D), jnp.float32),
                   jax.ShapeDtypeStruct((NP, C), jnp.float32)],
    )(acc_i, cnt_i, root_i, acc_u, cnt_u, root_u, wr1, b1)


def _tc2_body(acc_a, acc_b, cnt_u, root1, wn1, out):
    cu = jnp.maximum(cnt_u[...], 1.0)
    mean = (acc_a[...] + acc_b[...]) / cu
    out[...] = lax.dot_general(mean, wn1[...], _DN,
                               preferred_element_type=jnp.float32) + root1[...]


def _tc2(acc_a, acc_b, cnt_u, root1, wn1):
    row = pl.BlockSpec((_BLK, D), lambda i: (i, 0))
    blk = pl.BlockSpec((_BLK, C), lambda i: (i, 0))
    cnt = pl.BlockSpec((_BLK, 1), lambda i: (i, 0))
    w = pl.BlockSpec((C, D), lambda i: (0, 0))
    return pl.pallas_call(
        _tc2_body,
        grid=(NP // _BLK,),
        in_specs=[row, row, cnt, blk, w],
        out_specs=blk,
        out_shape=jax.ShapeDtypeStruct((NP, C), jnp.float32),
    )(acc_a, acc_b, cnt_u, root1, wn1)


def _fill2d(ref, nrows, nvecs, val):
    def body(i, _):
        for j in range(nvecs):
            ref[i, pl.ds(j * 16, 16)] = jnp.full((16,), val, jnp.float32)
        return 0
    lax.fori_loop(0, nrows, body, 0)


def _fill1d(ref, nvecs, val):
    def body(i, _):
        ref[pl.ds(i * 16, 16)] = jnp.full((16,), val, jnp.float32)
        return 0
    lax.fori_loop(0, nvecs, body, 0)


@functools.partial(
    pl.kernel,
    out_type=(
        jax.ShapeDtypeStruct((NP, D), jnp.float32),
        jax.ShapeDtypeStruct((NP,), jnp.float32),
        jax.ShapeDtypeStruct((NP, D), jnp.float32),
        jax.ShapeDtypeStruct((NP,), jnp.float32),
    ),
    mesh=_mesh,
    scratch_types=[
        pltpu.VMEM((HALF, CH), jnp.int32),
        pltpu.VMEM((HALF, CH), jnp.int32),
        pltpu.VMEM((2, CH, D), jnp.float32),
        pltpu.VMEM((CH,), jnp.float32),
        pltpu.VMEM((STRIPE,), jnp.float32),
        pltpu.VMEM_SHARED((NP, D), jnp.float32),
        pltpu.VMEM_SHARED((NP,), jnp.float32),
        pltpu.SemaphoreType.DMA,
        pltpu.SemaphoreType.DMA,
        pltpu.SemaphoreType.DMA,
    ],
)
def _sc_layer0(m_u2i, s_u2i, d_u2i, m_i2u, s_i2u, d_i2u,
               acc_i, cnt_i, acc_u, cnt_u,
               idx_s, idx_d, rows, ones, zcnt, acc_sh, cnt_sh,
               sem_g0, sem_g1, sem_c):
    cid = lax.axis_index("c")
    sid = lax.axis_index("s")
    base = sid * STRIPE

    _fill1d(ones, CH // 16, 1.0)
    _fill1d(zcnt, STRIPE // 16, 0.0)
    _fill2d(rows.at[0], CH, D // 16, 0.0)
    for k in range(STRIPE // CH):
        pltpu.sync_copy(rows.at[0], acc_sh.at[pl.ds(base + k * CH, CH)])
    pltpu.sync_copy(zcnt, cnt_sh.at[pl.ds(base, STRIPE)])
    plsc.subcore_barrier()

    def do_rel(msg, s2d, d2d):
        sems = (sem_g0, sem_g1)
        for half in range(2):
            rbase = sid * RPT + half * HALF
            pltpu.sync_copy(s2d.at[pl.ds(rbase, HALF)], idx_s)
            pltpu.sync_copy(d2d.at[pl.ds(rbase, HALF)], idx_d)
            for b in range(2):
                pltpu.make_async_copy(
                    msg.at[idx_s.at[b]], rows.at[b], sems[b]).start()

            def round_(r, _):
                cds = []
                for b in range(2):
                    j = 2 * r + b
                    pltpu.make_async_copy(
                        msg.at[idx_s.at[j]], rows.at[b], sems[b]).wait()
                    cds.append(pltpu.async_copy(
                        ones, cnt_sh.at[idx_d.at[j]], sem_c, add=True))
                    pltpu.sync_copy(rows.at[b], acc_sh.at[idx_d.at[j]],
                                    add=True)

                    @pl.when(j + 2 < HALF)
                    def _():
                        pltpu.make_async_copy(
                            msg.at[idx_s.at[j + 2]], rows.at[b],
                            sems[b]).start()
                for b in range(2):
                    cds[b].wait()
                return 0
            lax.fori_loop(0, HALF // 2, round_, 0)

    @pl.when(cid == 0)
    def _():
        do_rel(m_u2i, s_u2i, d_u2i)

    @pl.when(cid == 1)
    def _():
        do_rel(m_i2u, s_i2u, d_i2u)

    plsc.subcore_barrier()

    @pl.when(cid == 0)
    def _():
        pltpu.sync_copy(acc_sh.at[pl.ds(base, STRIPE)],
                        acc_i.at[pl.ds(base, STRIPE)])
        pltpu.sync_copy(cnt_sh.at[pl.ds(base, STRIPE)],
                        cnt_i.at[pl.ds(base, STRIPE)])

    @pl.when(cid == 1)
    def _():
        pltpu.sync_copy(acc_sh.at[pl.ds(base, STRIPE)],
                        acc_u.at[pl.ds(base, STRIPE)])
        pltpu.sync_copy(cnt_sh.at[pl.ds(base, STRIPE)],
                        cnt_u.at[pl.ds(base, STRIPE)])


@functools.partial(
    pl.kernel,
    out_type=(
        jax.ShapeDtypeStruct((NP, D), jnp.float32),
        jax.ShapeDtypeStruct((NP, D), jnp.float32),
    ),
    mesh=_mesh,
    scratch_types=[
        pltpu.VMEM((RPT2, CH), jnp.int32),
        pltpu.VMEM((RPT2, CH), jnp.int32),
        pltpu.VMEM((2, CH, D), jnp.float32),
        pltpu.VMEM_SHARED((NP, D), jnp.float32),
        pltpu.SemaphoreType.DMA,
        pltpu.SemaphoreType.DMA,
    ],
)
def _sc_layer1(msg1, s_i2u, d_i2u, acc_a, acc_b,
               idx_s, idx_d, rows, acc_sh, sem_g0, sem_g1):
    cid = lax.axis_index("c")
    sid = lax.axis_index("s")
    base = sid * STRIPE

    _fill2d(rows.at[0], CH, D // 16, 0.0)
    for k in range(STRIPE // CH):
        pltpu.sync_copy(rows.at[0], acc_sh.at[pl.ds(base + k * CH, CH)])
    plsc.subcore_barrier()

    rbase = (cid * NTILE + sid) * RPT2
    pltpu.sync_copy(s_i2u.at[pl.ds(rbase, RPT2)], idx_s)
    pltpu.sync_copy(d_i2u.at[pl.ds(rbase, RPT2)], idx_d)

    sems = (sem_g0, sem_g1)
    for b in range(2):
        pltpu.make_async_copy(
            msg1.at[idx_s.at[b]], rows.at[b], sems[b]).start()

    def round_(r, _):
        for b in range(2):
            j = 2 * r + b
            pltpu.make_async_copy(
                msg1.at[idx_s.at[j]], rows.at[b], sems[b]).wait()
            pltpu.sync_copy(rows.at[b], acc_sh.at[idx_d.at[j]], add=True)

            @pl.when(j + 2 < RPT2)
            def _():
                pltpu.make_async_copy(
                    msg1.at[idx_s.at[j + 2]], rows.at[b], sems[b]).start()
        return 0
    lax.fori_loop(0, RPT2 // 2, round_, 0)
    plsc.subcore_barrier()

    @pl.when(cid == 0)
    def _():
        pltpu.sync_copy(acc_sh.at[pl.ds(base, STRIPE)],
                        acc_a.at[pl.ds(base, STRIPE)])

    @pl.when(cid == 1)
    def _():
        pltpu.sync_copy(acc_sh.at[pl.ds(base, STRIPE)],
                        acc_b.at[pl.ds(base, STRIPE)])


def _pad_rows(x):
    return jnp.concatenate(
        [x, jnp.zeros((NP - N,) + x.shape[1:], x.dtype)], axis=0)


def _pad_edges(src, dst):
    ar = jnp.arange(EP - E, dtype=jnp.int32)
    s = jnp.concatenate([src, (ar * 37) % N]).reshape(ROWS, CH)
    d = jnp.concatenate([dst, N + (ar % (NP - N))]).reshape(ROWS, CH)
    return s, d


def kernel(x_user, x_item, Wn0_u2i, b0_u2i, Wr0_u2i, Wn0_i2u, b0_i2u, Wr0_i2u,
           Wn1_u2i, b1_u2i, Wr1_u2i, Wn1_i2u, b1_i2u, Wr1_i2u,
           ei_u2i, ei_i2u):
    s_u2i, d_u2i = _pad_edges(ei_u2i[0], ei_u2i[1])
    s_i2u, d_i2u = _pad_edges(ei_i2u[0], ei_i2u[1])

    m_u2i, r_i, m_i2u, r_u = _tc0(
        _pad_rows(x_user), _pad_rows(x_item),
        Wn0_u2i, Wr0_u2i, Wn0_i2u, Wr0_i2u,
        b0_u2i.reshape(1, D), b0_i2u.reshape(1, D))

    acc_i, cnt_i, acc_u, cnt_u = _sc_layer0(
        m_u2i, s_u2i, d_u2i, m_i2u, s_i2u, d_i2u)

    h_i, root1 = _tc1(acc_i, cnt_i.reshape(NP, 1), r_i,
                      acc_u, cnt_u.reshape(NP, 1), r_u,
                      Wr1_i2u, b1_i2u.reshape(1, C))

    acc_a, acc_b = _sc_layer1(h_i, s_i2u, d_i2u)

    return _tc2(acc_a, acc_b, cnt_u.reshape(NP, 1), root1, Wn1_i2u)[:N]


# speedup vs baseline: 10.7562x; 1.0546x over previous
import functools

import jax
import jax.numpy as jnp
from jax import lax
from jax.experimental import pallas as pl
from jax.experimental.pallas import tpu as pltpu
from jax.experimental.pallas import tpu_sc as plsc

N = 10000
NP = 10240
E = 160000
D = 128
C = 16
CH = 128
ROWS = 1280
EP = ROWS * CH
NSC = 2
NTILE = 16
RPT = ROWS // NTILE
HALF = RPT // 2
RPT2 = ROWS // (2 * NTILE)
STRIPE = NP // NTILE

_mesh = plsc.VectorSubcoreMesh(
    core_axis_name="c", subcore_axis_name="s",
    num_cores=NSC, num_subcores=NTILE)

_DN = (((1,), (1,)), ((), ()))

_BLK = 1024


def _tc0a_body(xu, xi, wn_u2i, wn_i2u, m_u2i, m_i2u):
    m_u2i[...] = lax.dot_general(xu[...], wn_u2i[...], _DN,
                                 preferred_element_type=jnp.float32)
    m_i2u[...] = lax.dot_general(xi[...], wn_i2u[...], _DN,
                                 preferred_element_type=jnp.float32)


def _tc0a(xu, xi, wn_u2i, wn_i2u):
    row = pl.BlockSpec((_BLK, D), lambda i: (i, 0))
    w = pl.BlockSpec((D, D), lambda i: (0, 0))
    return pl.pallas_call(
        _tc0a_body,
        grid=(NP // _BLK,),
        in_specs=[row, row, w, w],
        out_specs=[row, row],
        out_shape=[jax.ShapeDtypeStruct((NP, D), jnp.float32)] * 2,
    )(xu, xi, wn_u2i, wn_i2u)


def _tc0b_body(xu, xi, wr_u2i, wr_i2u, b_u2i, b_i2u, r_i, r_u):
    r_i[...] = lax.dot_general(xi[...], wr_u2i[...], _DN,
                               preferred_element_type=jnp.float32) + b_u2i[...]
    r_u[...] = lax.dot_general(xu[...], wr_i2u[...], _DN,
                               preferred_element_type=jnp.float32) + b_i2u[...]


def _tc0b(xu, xi, wr_u2i, wr_i2u, b_u2i, b_i2u):
    row = pl.BlockSpec((_BLK, D), lambda i: (i, 0))
    w = pl.BlockSpec((D, D), lambda i: (0, 0))
    b = pl.BlockSpec((1, D), lambda i: (0, 0))
    return pl.pallas_call(
        _tc0b_body,
        grid=(NP // _BLK,),
        in_specs=[row, row, w, w, b, b],
        out_specs=[row, row],
        out_shape=[jax.ShapeDtypeStruct((NP, D), jnp.float32)] * 2,
    )(xu, xi, wr_u2i, wr_i2u, b_u2i, b_i2u)


def _tc1a_body(acc_i, cnt_i, root_i, h_i):
    ci = jnp.maximum(cnt_i[...], 1.0)
    h_i[...] = jnp.maximum(acc_i[...] / ci + root_i[...], 0.0)


def _tc1a(acc_i, cnt_i, root_i):
    row = pl.BlockSpec((_BLK, D), lambda i: (i, 0))
    cnt = pl.BlockSpec((_BLK, 1), lambda i: (i, 0))
    return pl.pallas_call(
        _tc1a_body,
        grid=(NP // _BLK,),
        in_specs=[row, cnt, row],
        out_specs=row,
        out_shape=jax.ShapeDtypeStruct((NP, D), jnp.float32),
    )(acc_i, cnt_i, root_i)


def _tc1b_body(acc_u, cnt_u, root_u, wr1, b1, root1):
    cu = jnp.maximum(cnt_u[...], 1.0)
    h_u = jnp.maximum(acc_u[...] / cu + root_u[...], 0.0)
    root1[...] = lax.dot_general(h_u, wr1[...], _DN,
                                 preferred_element_type=jnp.float32) + b1[...]


def _tc1b(acc_u, cnt_u, root_u, wr1, b1):
    row = pl.BlockSpec((_BLK, D), lambda i: (i, 0))
    cnt = pl.BlockSpec((_BLK, 1), lambda i: (i, 0))
    w = pl.BlockSpec((C, D), lambda i: (0, 0))
    b = pl.BlockSpec((1, C), lambda i: (0, 0))
    out = pl.BlockSpec((_BLK, C), lambda i: (i, 0))
    return pl.pallas_call(
        _tc1b_body,
        grid=(NP // _BLK,),
        in_specs=[row, cnt, row, w, b],
        out_specs=out,
        out_shape=jax.ShapeDtypeStruct((NP, C), jnp.float32),
    )(acc_u, cnt_u, root_u, wr1, b1)


def _tc2_body(acc_a, acc_b, cnt_u, root1, wn1, out):
    cu = jnp.maximum(cnt_u[...], 1.0)
    mean = (acc_a[...] + acc_b[...]) / cu
    out[...] = lax.dot_general(mean, wn1[...], _DN,
                               preferred_element_type=jnp.float32) + root1[...]


def _tc2(acc_a, acc_b, cnt_u, root1, wn1):
    row = pl.BlockSpec((_BLK, D), lambda i: (i, 0))
    blk = pl.BlockSpec((_BLK, C), lambda i: (i, 0))
    cnt = pl.BlockSpec((_BLK, 1), lambda i: (i, 0))
    w = pl.BlockSpec((C, D), lambda i: (0, 0))
    return pl.pallas_call(
        _tc2_body,
        grid=(NP // _BLK,),
        in_specs=[row, row, cnt, blk, w],
        out_specs=blk,
        out_shape=jax.ShapeDtypeStruct((NP, C), jnp.float32),
    )(acc_a, acc_b, cnt_u, root1, wn1)


def _fill2d(ref, nrows, nvecs, val):
    def body(i, _):
        for j in range(nvecs):
            ref[i, pl.ds(j * 16, 16)] = jnp.full((16,), val, jnp.float32)
        return 0
    lax.fori_loop(0, nrows, body, 0)


def _fill1d(ref, nvecs, val):
    def body(i, _):
        ref[pl.ds(i * 16, 16)] = jnp.full((16,), val, jnp.float32)
        return 0
    lax.fori_loop(0, nvecs, body, 0)


@functools.partial(
    pl.kernel,
    out_type=(
        jax.ShapeDtypeStruct((NP, D), jnp.float32),
        jax.ShapeDtypeStruct((NP,), jnp.float32),
        jax.ShapeDtypeStruct((NP, D), jnp.float32),
        jax.ShapeDtypeStruct((NP,), jnp.float32),
    ),
    mesh=_mesh,
    scratch_types=[
        pltpu.VMEM((HALF, CH), jnp.int32),
        pltpu.VMEM((HALF, CH), jnp.int32),
        pltpu.VMEM((2, CH, D), jnp.float32),
        pltpu.VMEM((CH,), jnp.float32),
        pltpu.VMEM((STRIPE,), jnp.float32),
        pltpu.VMEM_SHARED((NP, D), jnp.float32),
        pltpu.VMEM_SHARED((NP,), jnp.float32),
        pltpu.SemaphoreType.DMA,
        pltpu.SemaphoreType.DMA,
        pltpu.SemaphoreType.DMA,
    ],
)
def _sc_layer0(m_u2i, s_u2i, d_u2i, m_i2u, s_i2u, d_i2u,
               acc_i, cnt_i, acc_u, cnt_u,
               idx_s, idx_d, rows, ones, zcnt, acc_sh, cnt_sh,
               sem_g0, sem_g1, sem_c):
    cid = lax.axis_index("c")
    sid = lax.axis_index("s")
    base = sid * STRIPE

    _fill1d(ones, CH // 16, 1.0)
    _fill1d(zcnt, STRIPE // 16, 0.0)
    _fill2d(rows.at[0], CH, D // 16, 0.0)
    for k in range(STRIPE // CH):
        pltpu.sync_copy(rows.at[0], acc_sh.at[pl.ds(base + k * CH, CH)])
    pltpu.sync_copy(zcnt, cnt_sh.at[pl.ds(base, STRIPE)])
    plsc.subcore_barrier()

    def do_rel(msg, s2d, d2d):
        sems = (sem_g0, sem_g1)
        for half in range(2):
            rbase = sid * RPT + half * HALF
            pltpu.sync_copy(s2d.at[pl.ds(rbase, HALF)], idx_s)
            pltpu.sync_copy(d2d.at[pl.ds(rbase, HALF)], idx_d)
            for b in range(2):
                pltpu.make_async_copy(
                    msg.at[idx_s.at[b]], rows.at[b], sems[b]).start()

            def round_(r, _):
                cds = []
                for b in range(2):
                    j = 2 * r + b
                    pltpu.make_async_copy(
                        msg.at[idx_s.at[j]], rows.at[b], sems[b]).wait()
                    cds.append(pltpu.async_copy(
                        ones, cnt_sh.at[idx_d.at[j]], sem_c, add=True))
                    pltpu.sync_copy(rows.at[b], acc_sh.at[idx_d.at[j]],
                                    add=True)

                    @pl.when(j + 2 < HALF)
                    def _():
                        pltpu.make_async_copy(
                            msg.at[idx_s.at[j + 2]], rows.at[b],
                            sems[b]).start()
                for b in range(2):
                    cds[b].wait()
                return 0
            lax.fori_loop(0, HALF // 2, round_, 0)

    @pl.when(cid == 0)
    def _():
        do_rel(m_u2i, s_u2i, d_u2i)

    @pl.when(cid == 1)
    def _():
        do_rel(m_i2u, s_i2u, d_i2u)

    plsc.subcore_barrier()

    @pl.when(cid == 0)
    def _():
        pltpu.sync_copy(acc_sh.at[pl.ds(base, STRIPE)],
                        acc_i.at[pl.ds(base, STRIPE)])
        pltpu.sync_copy(cnt_sh.at[pl.ds(base, STRIPE)],
                        cnt_i.at[pl.ds(base, STRIPE)])

    @pl.when(cid == 1)
    def _():
        pltpu.sync_copy(acc_sh.at[pl.ds(base, STRIPE)],
                        acc_u.at[pl.ds(base, STRIPE)])
        pltpu.sync_copy(cnt_sh.at[pl.ds(base, STRIPE)],
                        cnt_u.at[pl.ds(base, STRIPE)])


@functools.partial(
    pl.kernel,
    out_type=(
        jax.ShapeDtypeStruct((NP, D), jnp.float32),
        jax.ShapeDtypeStruct((NP, D), jnp.float32),
    ),
    mesh=_mesh,
    scratch_types=[
        pltpu.VMEM((RPT2, CH), jnp.int32),
        pltpu.VMEM((RPT2, CH), jnp.int32),
        pltpu.VMEM((2, CH, D), jnp.float32),
        pltpu.VMEM_SHARED((NP, D), jnp.float32),
        pltpu.SemaphoreType.DMA,
        pltpu.SemaphoreType.DMA,
    ],
)
def _sc_layer1(msg1, s_i2u, d_i2u, acc_a, acc_b,
               idx_s, idx_d, rows, acc_sh, sem_g0, sem_g1):
    cid = lax.axis_index("c")
    sid = lax.axis_index("s")
    base = sid * STRIPE

    _fill2d(rows.at[0], CH, D // 16, 0.0)
    for k in range(STRIPE // CH):
        pltpu.sync_copy(rows.at[0], acc_sh.at[pl.ds(base + k * CH, CH)])
    plsc.subcore_barrier()

    rbase = (cid * NTILE + sid) * RPT2
    pltpu.sync_copy(s_i2u.at[pl.ds(rbase, RPT2)], idx_s)
    pltpu.sync_copy(d_i2u.at[pl.ds(rbase, RPT2)], idx_d)

    sems = (sem_g0, sem_g1)
    for b in range(2):
        pltpu.make_async_copy(
            msg1.at[idx_s.at[b]], rows.at[b], sems[b]).start()

    def round_(r, _):
        for b in range(2):
            j = 2 * r + b
            pltpu.make_async_copy(
                msg1.at[idx_s.at[j]], rows.at[b], sems[b]).wait()
            pltpu.sync_copy(rows.at[b], acc_sh.at[idx_d.at[j]], add=True)

            @pl.when(j + 2 < RPT2)
            def _():
                pltpu.make_async_copy(
                    msg1.at[idx_s.at[j + 2]], rows.at[b], sems[b]).start()
        return 0
    lax.fori_loop(0, RPT2 // 2, round_, 0)
    plsc.subcore_barrier()

    @pl.when(cid == 0)
    def _():
        pltpu.sync_copy(acc_sh.at[pl.ds(base, STRIPE)],
                        acc_a.at[pl.ds(base, STRIPE)])

    @pl.when(cid == 1)
    def _():
        pltpu.sync_copy(acc_sh.at[pl.ds(base, STRIPE)],
                        acc_b.at[pl.ds(base, STRIPE)])


def _pad_rows(x):
    return jnp.concatenate(
        [x, jnp.zeros((NP - N,) + x.shape[1:], x.dtype)], axis=0)


def _pad_edges(src, dst):
    ar = jnp.arange(EP - E, dtype=jnp.int32)
    s = jnp.concatenate([src, (ar * 37) % N]).reshape(ROWS, CH)
    d = jnp.concatenate([dst, N + (ar % (NP - N))]).reshape(ROWS, CH)
    return s, d


def kernel(x_user, x_item, Wn0_u2i, b0_u2i, Wr0_u2i, Wn0_i2u, b0_i2u, Wr0_i2u,
           Wn1_u2i, b1_u2i, Wr1_u2i, Wn1_i2u, b1_i2u, Wr1_i2u,
           ei_u2i, ei_i2u):
    s_u2i, d_u2i = _pad_edges(ei_u2i[0], ei_u2i[1])
    s_i2u, d_i2u = _pad_edges(ei_i2u[0], ei_i2u[1])

    m_u2i, m_i2u = _tc0a(x_user, x_item, Wn0_u2i, Wn0_i2u)
    r_i, r_u = _tc0b(x_user, x_item, Wr0_u2i, Wr0_i2u,
                     b0_u2i.reshape(1, D), b0_i2u.reshape(1, D))

    acc_i, cnt_i, acc_u, cnt_u = _sc_layer0(
        m_u2i, s_u2i, d_u2i, m_i2u, s_i2u, d_i2u)

    h_i = _tc1a(acc_i, cnt_i.reshape(NP, 1), r_i)
    root1 = _tc1b(acc_u, cnt_u.reshape(NP, 1), r_u,
                  Wr1_i2u, b1_i2u.reshape(1, C))

    acc_a, acc_b = _sc_layer1(h_i, s_i2u, d_i2u)

    return _tc2(acc_a, acc_b, cnt_u.reshape(NP, 1), root1, Wn1_i2u)[:N]
